# baseline (speedup 1.0000x reference)
import functools

import jax
import jax.numpy as jnp
from jax import lax
from jax.experimental import pallas as pl
from jax.experimental.pallas import tpu as pltpu
from jax.experimental.pallas import tpu_sc as plsc

N = 10000
E = 320000
D = 128
ED = 4
L = 3

NUM_SC = 2
NUM_TILES = 16
ROWS_MAIN = 632
ROWS_LAST = N - (NUM_TILES - 1) * ROWS_MAIN
LAST_ROW0 = (NUM_TILES - 1) * ROWS_MAIN
EDGES_PER_TILE = E // (NUM_SC * NUM_TILES)
CHUNK = 40
NCHUNK = EDGES_PER_TILE // CHUNK
NBUF = 3

ROW_BLK = 400
GRID_N = N // ROW_BLK
E_BLK = 2000
GRID_E = E // E_BLK

_sc_mesh = plsc.VectorSubcoreMesh(core_axis_name="c", subcore_axis_name="s")


def _edge_body(pre_d_hbm, pre_s_hbm, ea_hbm, dst_hbm, src_hbm,
               zeros_hbm, out_hbm, *scr):
    idx_d = scr[0:3]
    idx_s = scr[3:6]
    gd = scr[6:9]
    gs = scr[9:12]
    ge = scr[12:15]
    acc = scr[15]
    sem_i = scr[16:19]
    sem_g = scr[19:22]

    c = lax.axis_index("c")
    s = lax.axis_index("s")
    row0 = s * ROWS_MAIN

    @pl.when(s < NUM_TILES - 1)
    def _():
        pltpu.sync_copy(zeros_hbm.at[pl.ds(row0, ROWS_MAIN)],
                        acc.at[pl.ds(row0, ROWS_MAIN)])

    @pl.when(s == NUM_TILES - 1)
    def _():
        pltpu.sync_copy(zeros_hbm.at[pl.ds(LAST_ROW0, ROWS_LAST)],
                        acc.at[pl.ds(LAST_ROW0, ROWS_LAST)])

    plsc.subcore_barrier()

    edge0 = c * (E // NUM_SC) + s * EDGES_PER_TILE

    def idx_copies(i, b):
        base = edge0 + i * CHUNK
        return (pltpu.make_async_copy(dst_hbm.at[pl.ds(base, CHUNK)],
                                      idx_d[b], sem_i[b]),
                pltpu.make_async_copy(src_hbm.at[pl.ds(base, CHUNK)],
                                      idx_s[b], sem_i[b]))

    def gather_copies(i, b):
        base = edge0 + i * CHUNK
        return (pltpu.make_async_copy(pre_d_hbm.at[idx_d[b]], gd[b],
                                      sem_g[b]),
                pltpu.make_async_copy(pre_s_hbm.at[idx_s[b]], gs[b],
                                      sem_g[b]),
                pltpu.make_async_copy(ea_hbm.at[pl.ds(base, CHUNK)], ge[b],
                                      sem_g[b]))

    def step(i, b, g1, ix2):
        b1 = (b + 1) % NBUF
        b2 = (b + 2) % NBUF
        if g1:
            for cp in idx_copies(i + 1, b1):
                cp.wait()
            for cp in gather_copies(i + 1, b1):
                cp.start()
        if ix2:
            for cp in idx_copies(i + 2, b2):
                cp.start()
        for cp in gather_copies(i, b):
            cp.wait()

        @pl.loop(0, CHUNK)
        def _(e):
            for cc in range(D // 16):
                slc = (pl.ds(e, 1), pl.ds(cc * 16, 16))
                v = gd[b].at[slc][...] + gs[b].at[slc][...] \
                    + ge[b].at[slc][...]
                gd[b].at[slc][...] = jnp.maximum(v, 0.0)

        pltpu.sync_copy(gd[b], acc.at[idx_d[b]], add=True)

    for cp in idx_copies(0, 0):
        cp.start()
    for cp in idx_copies(0, 0):
        cp.wait()
    for cp in gather_copies(0, 0):
        cp.start()
    for cp in idx_copies(1, 1):
        cp.start()

    step(0, 0, g1=True, ix2=True)
    step(1, 1, g1=True, ix2=True)
    step(2, 2, g1=True, ix2=True)

    @pl.loop(3, NCHUNK - 4, step=NBUF)
    def _(i):
        step(i, 0, g1=True, ix2=True)
        step(i + 1, 1, g1=True, ix2=True)
        step(i + 2, 2, g1=True, ix2=True)

    step(NCHUNK - 4, 0, g1=True, ix2=True)
    step(NCHUNK - 3, 1, g1=True, ix2=True)
    step(NCHUNK - 2, 2, g1=True, ix2=False)
    step(NCHUNK - 1, 0, g1=False, ix2=False)

    plsc.subcore_barrier()

    @pl.when(s < NUM_TILES - 1)
    def _():
        pltpu.sync_copy(acc.at[pl.ds(row0, ROWS_MAIN)],
                        out_hbm.at[c].at[pl.ds(row0, ROWS_MAIN)])

    @pl.when(s == NUM_TILES - 1)
    def _():
        pltpu.sync_copy(acc.at[pl.ds(LAST_ROW0, ROWS_LAST)],
                        out_hbm.at[c].at[pl.ds(LAST_ROW0, ROWS_LAST)])


def _edge_pass(pre_d, pre_s, ea_pre, dst, src, zeros_nd):
    k = pl.kernel(
        _edge_body,
        out_type=jax.ShapeDtypeStruct((NUM_SC, N, D), jnp.float32),
        mesh=_sc_mesh,
        scratch_types=(
            [pltpu.VMEM((CHUNK,), jnp.int32)] * (2 * NBUF)
            + [pltpu.VMEM((CHUNK, D), jnp.float32)] * (3 * NBUF)
            + [pltpu.VMEM_SHARED((N, D), jnp.float32)]
            + [pltpu.SemaphoreType.DMA] * (2 * NBUF)
        ),
    )
    return k(pre_d, pre_s, ea_pre, dst, src, zeros_nd)


def _count_body(dst_hbm, zeros_hbm, out_hbm, idx_d, ones, acc):
    c = lax.axis_index("c")
    s = lax.axis_index("s")
    row0 = s * ROWS_MAIN

    @pl.when(s < NUM_TILES - 1)
    def _():
        pltpu.sync_copy(zeros_hbm.at[pl.ds(row0, ROWS_MAIN)],
                        acc.at[pl.ds(row0, ROWS_MAIN)])

    @pl.when(s == NUM_TILES - 1)
    def _():
        pltpu.sync_copy(zeros_hbm.at[pl.ds(LAST_ROW0, ROWS_LAST)],
                        acc.at[pl.ds(LAST_ROW0, ROWS_LAST)])

    @pl.loop(0, CHUNK)
    def _(e):
        ones.at[pl.ds(e, 1), pl.ds(0, 16)][...] = jnp.full((1, 16), 1.0,
                                                           jnp.float32)
    plsc.subcore_barrier()

    edge0 = c * (E // NUM_SC) + s * EDGES_PER_TILE

    @pl.loop(0, NCHUNK)
    def _(chunk):
        base = edge0 + chunk * CHUNK
        pltpu.sync_copy(dst_hbm.at[pl.ds(base, CHUNK)], idx_d)
        pltpu.sync_copy(ones, acc.at[idx_d], add=True)

    plsc.subcore_barrier()

    @pl.when(s < NUM_TILES - 1)
    def _():
        pltpu.sync_copy(acc.at[pl.ds(row0, ROWS_MAIN)],
                        out_hbm.at[c].at[pl.ds(row0, ROWS_MAIN)])

    @pl.when(s == NUM_TILES - 1)
    def _():
        pltpu.sync_copy(acc.at[pl.ds(LAST_ROW0, ROWS_LAST)],
                        out_hbm.at[c].at[pl.ds(LAST_ROW0, ROWS_LAST)])


def _count_pass(dst, zeros_n16):
    k = pl.kernel(
        _count_body,
        out_type=jax.ShapeDtypeStruct((NUM_SC, N, 16), jnp.float32),
        mesh=_sc_mesh,
        scratch_types=[
            pltpu.VMEM((CHUNK,), jnp.int32),
            pltpu.VMEM((CHUNK, 16), jnp.float32),
            pltpu.VMEM_SHARED((N, 16), jnp.float32),
        ],
    )
    return k(dst, zeros_n16)


def _dot(a, b):
    return jnp.dot(a, b, preferred_element_type=jnp.float32)


def _in_proj_body(x_ref, w_ref, b_ref, o_ref):
    o_ref[...] = jnp.maximum(_dot(x_ref[...], w_ref[...]) + b_ref[...], 0.0)


def _in_proj(x, w, b):
    return pl.pallas_call(
        _in_proj_body,
        grid=(GRID_N,),
        in_specs=[
            pl.BlockSpec((ROW_BLK, D), lambda i: (i, 0)),
            pl.BlockSpec((D, D), lambda i: (0, 0)),
            pl.BlockSpec((1, D), lambda i: (0, 0)),
        ],
        out_specs=pl.BlockSpec((ROW_BLK, D), lambda i: (i, 0)),
        out_shape=jax.ShapeDtypeStruct((N, D), jnp.float32),
    )(x, w, b)


def _pre_proj_body(h_ref, wd_ref, ws_ref, pd_ref, ps_ref):
    h = h_ref[...]
    pd_ref[...] = _dot(h, wd_ref[...])
    ps_ref[...] = _dot(h, ws_ref[...])


def _pre_proj(h, wd, ws):
    return pl.pallas_call(
        _pre_proj_body,
        grid=(GRID_N,),
        in_specs=[
            pl.BlockSpec((ROW_BLK, D), lambda i: (i, 0)),
            pl.BlockSpec((D, D), lambda i: (0, 0)),
            pl.BlockSpec((D, D), lambda i: (0, 0)),
        ],
        out_specs=[
            pl.BlockSpec((ROW_BLK, D), lambda i: (i, 0)),
            pl.BlockSpec((ROW_BLK, D), lambda i: (i, 0)),
        ],
        out_shape=[
            jax.ShapeDtypeStruct((N, D), jnp.float32),
            jax.ShapeDtypeStruct((N, D), jnp.float32),
        ],
    )(h, wd, ws)


def _ea_proj_body(ea_ref, w_ref, b_ref, o_ref):
    o_ref[...] = _dot(ea_ref[...], w_ref[...]) + b_ref[...]


def _ea_proj(edge_attr, w_e, b1):
    return pl.pallas_call(
        _ea_proj_body,
        grid=(GRID_E,),
        in_specs=[
            pl.BlockSpec((E_BLK, ED), lambda i: (i, 0)),
            pl.BlockSpec((ED, D), lambda i: (0, 0)),
            pl.BlockSpec((1, D), lambda i: (0, 0)),
        ],
        out_specs=pl.BlockSpec((E_BLK, D), lambda i: (i, 0)),
        out_shape=jax.ShapeDtypeStruct((E, D), jnp.float32),
    )(edge_attr, w_e, b1)


def _update_body(a0_ref, a1_ref, c0_ref, c1_ref, h_ref, w2_ref, b2_ref,
                 u1a_ref, u1b_ref, ub1_ref, u2_ref, ub2_ref, g_ref, b_ref,
                 o_ref):
    h = h_ref[...]
    agg_raw = a0_ref[0] + a1_ref[0]
    cnt = c0_ref[...] + c1_ref[...]
    aggr = _dot(agg_raw, w2_ref[...]) + cnt * b2_ref[...]
    t = jnp.maximum(_dot(h, u1a_ref[...]) + _dot(aggr, u1b_ref[...])
                    + ub1_ref[...], 0.0)
    t2 = _dot(t, u2_ref[...]) + ub2_ref[...]
    mu = jnp.mean(t2, axis=1, keepdims=True)
    var = jnp.mean((t2 - mu) ** 2, axis=1, keepdims=True)
    y = (t2 - mu) * lax.rsqrt(var + 1e-5) * g_ref[...] + b_ref[...]
    o_ref[...] = jnp.maximum(y, 0.0) + h


def _update(acc2, cnt0, cnt1, h, w2, b2, u1a, u1b, ub1, u2, ub2, g, b):
    row = lambda i: (i, 0)
    full = lambda i: (0, 0)
    return pl.pallas_call(
        _update_body,
        grid=(GRID_N,),
        in_specs=[
            pl.BlockSpec((1, ROW_BLK, D), lambda i: (0, i, 0)),
            pl.BlockSpec((1, ROW_BLK, D), lambda i: (1, i, 0)),
            pl.BlockSpec((ROW_BLK, 1), row),
            pl.BlockSpec((ROW_BLK, 1), row),
            pl.BlockSpec((ROW_BLK, D), row),
            pl.BlockSpec((D, D), full),
            pl.BlockSpec((1, D), full),
            pl.BlockSpec((D, D), full),
            pl.BlockSpec((D, D), full),
            pl.BlockSpec((1, D), full),
            pl.BlockSpec((D, D), full),
            pl.BlockSpec((1, D), full),
            pl.BlockSpec((1, D), full),
            pl.BlockSpec((1, D), full),
        ],
        out_specs=pl.BlockSpec((ROW_BLK, D), row),
        out_shape=jax.ShapeDtypeStruct((N, D), jnp.float32),
    )(acc2, acc2, cnt0, cnt1, h, w2, b2, u1a, u1b, ub1, u2, ub2, g, b)


def _out_proj_body(h_ref, w_ref, b_ref, o_ref):
    o_ref[...] = _dot(h_ref[...], w_ref[...]) + b_ref[...]


def _out_proj(h, w, b):
    return pl.pallas_call(
        _out_proj_body,
        grid=(GRID_N,),
        in_specs=[
            pl.BlockSpec((ROW_BLK, D), lambda i: (i, 0)),
            pl.BlockSpec((D, D), lambda i: (0, 0)),
            pl.BlockSpec((1, D), lambda i: (0, 0)),
        ],
        out_specs=pl.BlockSpec((ROW_BLK, D), lambda i: (i, 0)),
        out_shape=jax.ShapeDtypeStruct((N, D), jnp.float32),
    )(h, w, b)


def kernel(x, edge_index, edge_attr, in_w, in_b, msg_w1, msg_b1, msg_w2,
           msg_b2, upd_w1, upd_b1, upd_w2, upd_b2, ln_g, ln_b, out_w, out_b):
    src = edge_index[0]
    dst = edge_index[1]

    zeros_nd = jnp.zeros((N, D), jnp.float32)
    zeros_n16 = jnp.zeros((N, 16), jnp.float32)

    h = _in_proj(x, in_w, in_b.reshape(1, D))

    ea_pre = [_ea_proj(edge_attr, msg_w1[i, 2 * D:, :],
                       msg_b1[i].reshape(1, D)) for i in range(L)]

    cnt2 = _count_pass(dst, zeros_n16)
    cnt0 = cnt2[0, :, 0:1]
    cnt1 = cnt2[1, :, 0:1]

    for i in range(L):
        pre_d, pre_s = _pre_proj(h, msg_w1[i, :D, :], msg_w1[i, D:2 * D, :])
        acc2 = _edge_pass(pre_d, pre_s, ea_pre[i], dst, src, zeros_nd)
        h = _update(acc2, cnt0, cnt1, h,
                    msg_w2[i], msg_b2[i].reshape(1, D),
                    upd_w1[i, :D, :], upd_w1[i, D:, :],
                    upd_b1[i].reshape(1, D),
                    upd_w2[i], upd_b2[i].reshape(1, D),
                    ln_g[i].reshape(1, D), ln_b[i].reshape(1, D))

    return _out_proj(h, out_w, out_b.reshape(1, D))


# speedup vs baseline: 1.0298x; 1.0298x over previous
import functools

import jax
import jax.numpy as jnp
from jax import lax
from jax.experimental import pallas as pl
from jax.experimental.pallas import tpu as pltpu
from jax.experimental.pallas import tpu_sc as plsc

N = 10000
E = 320000
D = 128
ED = 4
L = 3

NUM_SC = 2
NUM_TILES = 16
ROWS_MAIN = 632
ROWS_LAST = N - (NUM_TILES - 1) * ROWS_MAIN
LAST_ROW0 = (NUM_TILES - 1) * ROWS_MAIN
EDGES_PER_TILE = E // (NUM_SC * NUM_TILES)
CHUNK = 40
NCHUNK = EDGES_PER_TILE // CHUNK
NBUF = 3

ROW_BLK = 400
GRID_N = N // ROW_BLK
E_BLK = 2000
GRID_E = E // E_BLK

_sc_mesh = plsc.VectorSubcoreMesh(core_axis_name="c", subcore_axis_name="s")


def _edge_body(pre_d_hbm, pre_s_hbm, ea_hbm, dst_hbm, src_hbm,
               zeros_hbm, out_hbm, *scr):
    idx_d = scr[0:3]
    idx_s = scr[3:6]
    gd = scr[6:9]
    gs = scr[9:12]
    ge = scr[12:15]
    acc = scr[15]
    sem_i = scr[16:19]
    sem_g = scr[19:22]

    c = lax.axis_index("c")
    s = lax.axis_index("s")
    row0 = s * ROWS_MAIN

    @pl.when(s < NUM_TILES - 1)
    def _():
        pltpu.sync_copy(zeros_hbm.at[pl.ds(row0, ROWS_MAIN)],
                        acc.at[pl.ds(row0, ROWS_MAIN)])

    @pl.when(s == NUM_TILES - 1)
    def _():
        pltpu.sync_copy(zeros_hbm.at[pl.ds(LAST_ROW0, ROWS_LAST)],
                        acc.at[pl.ds(LAST_ROW0, ROWS_LAST)])

    plsc.subcore_barrier()

    edge0 = c * (E // NUM_SC) + s * EDGES_PER_TILE

    def idx_copies(i, b):
        base = edge0 + i * CHUNK
        return (pltpu.make_async_copy(dst_hbm.at[pl.ds(base, CHUNK)],
                                      idx_d[b], sem_i[b]),
                pltpu.make_async_copy(src_hbm.at[pl.ds(base, CHUNK)],
                                      idx_s[b], sem_i[b]))

    def gather_copies(i, b):
        base = edge0 + i * CHUNK
        return (pltpu.make_async_copy(pre_d_hbm.at[idx_d[b]], gd[b],
                                      sem_g[b]),
                pltpu.make_async_copy(pre_s_hbm.at[idx_s[b]], gs[b],
                                      sem_g[b]),
                pltpu.make_async_copy(ea_hbm.at[pl.ds(base, CHUNK)], ge[b],
                                      sem_g[b]))

    def step(i, b, g1, ix2):
        b1 = (b + 1) % NBUF
        b2 = (b + 2) % NBUF
        if g1:
            for cp in idx_copies(i + 1, b1):
                cp.wait()
            for cp in gather_copies(i + 1, b1):
                cp.start()
        if ix2:
            for cp in idx_copies(i + 2, b2):
                cp.start()
        for cp in gather_copies(i, b):
            cp.wait()

        @pl.loop(0, CHUNK)
        def _(e):
            for cc in range(D // 16):
                slc = (pl.ds(e, 1), pl.ds(cc * 16, 16))
                v = gd[b].at[slc][...] + gs[b].at[slc][...] \
                    + ge[b].at[slc][...]
                gd[b].at[slc][...] = jnp.maximum(v, 0.0)

        pltpu.sync_copy(gd[b], acc.at[idx_d[b]], add=True)

    for cp in idx_copies(0, 0):
        cp.start()
    for cp in idx_copies(0, 0):
        cp.wait()
    for cp in gather_copies(0, 0):
        cp.start()
    for cp in idx_copies(1, 1):
        cp.start()

    step(0, 0, g1=True, ix2=True)
    step(1, 1, g1=True, ix2=True)
    step(2, 2, g1=True, ix2=True)

    @pl.loop(3, NCHUNK - 4, step=NBUF)
    def _(i):
        step(i, 0, g1=True, ix2=True)
        step(i + 1, 1, g1=True, ix2=True)
        step(i + 2, 2, g1=True, ix2=True)

    step(NCHUNK - 4, 0, g1=True, ix2=True)
    step(NCHUNK - 3, 1, g1=True, ix2=True)
    step(NCHUNK - 2, 2, g1=True, ix2=False)
    step(NCHUNK - 1, 0, g1=False, ix2=False)

    plsc.subcore_barrier()

    @pl.when(s < NUM_TILES - 1)
    def _():
        pltpu.sync_copy(acc.at[pl.ds(row0, ROWS_MAIN)],
                        out_hbm.at[c].at[pl.ds(row0, ROWS_MAIN)])

    @pl.when(s == NUM_TILES - 1)
    def _():
        pltpu.sync_copy(acc.at[pl.ds(LAST_ROW0, ROWS_LAST)],
                        out_hbm.at[c].at[pl.ds(LAST_ROW0, ROWS_LAST)])


def _edge_pass(pre_d, pre_s, ea_pre, dst, src, zeros_nd):
    k = pl.kernel(
        _edge_body,
        out_type=jax.ShapeDtypeStruct((NUM_SC, N, D), jnp.float32),
        mesh=_sc_mesh,
        scratch_types=(
            [pltpu.VMEM((CHUNK,), jnp.int32)] * (2 * NBUF)
            + [pltpu.VMEM((CHUNK, D), jnp.float32)] * (3 * NBUF)
            + [pltpu.VMEM_SHARED((N, D), jnp.float32)]
            + [pltpu.SemaphoreType.DMA] * (2 * NBUF)
        ),
    )
    return k(pre_d, pre_s, ea_pre, dst, src, zeros_nd)


def _count_body(dst_hbm, zeros_hbm, out_hbm, idx_d0, idx_d1, idx_d2, ones,
                acc, sem_i0, sem_i1, sem_i2):
    idx_d = (idx_d0, idx_d1, idx_d2)
    sem_i = (sem_i0, sem_i1, sem_i2)
    c = lax.axis_index("c")
    s = lax.axis_index("s")
    row0 = s * ROWS_MAIN

    @pl.when(s < NUM_TILES - 1)
    def _():
        pltpu.sync_copy(zeros_hbm.at[pl.ds(row0, ROWS_MAIN)],
                        acc.at[pl.ds(row0, ROWS_MAIN)])

    @pl.when(s == NUM_TILES - 1)
    def _():
        pltpu.sync_copy(zeros_hbm.at[pl.ds(LAST_ROW0, ROWS_LAST)],
                        acc.at[pl.ds(LAST_ROW0, ROWS_LAST)])

    @pl.loop(0, CHUNK)
    def _(e):
        ones.at[pl.ds(e, 1), pl.ds(0, 16)][...] = jnp.full((1, 16), 1.0,
                                                           jnp.float32)
    plsc.subcore_barrier()

    edge0 = c * (E // NUM_SC) + s * EDGES_PER_TILE

    def idx_copy(i, b):
        base = edge0 + i * CHUNK
        return pltpu.make_async_copy(dst_hbm.at[pl.ds(base, CHUNK)],
                                     idx_d[b], sem_i[b])

    def step(i, b, ix2):
        if ix2:
            idx_copy(i + 2, (b + 2) % NBUF).start()
        idx_copy(i, b).wait()
        pltpu.sync_copy(ones, acc.at[idx_d[b]], add=True)

    idx_copy(0, 0).start()
    idx_copy(1, 1).start()

    step(0, 0, ix2=True)
    step(1, 1, ix2=True)
    step(2, 2, ix2=True)

    @pl.loop(3, NCHUNK - 4, step=NBUF)
    def _(i):
        step(i, 0, ix2=True)
        step(i + 1, 1, ix2=True)
        step(i + 2, 2, ix2=True)

    step(NCHUNK - 4, 0, ix2=True)
    step(NCHUNK - 3, 1, ix2=True)
    step(NCHUNK - 2, 2, ix2=False)
    step(NCHUNK - 1, 0, ix2=False)

    plsc.subcore_barrier()

    @pl.when(s < NUM_TILES - 1)
    def _():
        pltpu.sync_copy(acc.at[pl.ds(row0, ROWS_MAIN)],
                        out_hbm.at[c].at[pl.ds(row0, ROWS_MAIN)])

    @pl.when(s == NUM_TILES - 1)
    def _():
        pltpu.sync_copy(acc.at[pl.ds(LAST_ROW0, ROWS_LAST)],
                        out_hbm.at[c].at[pl.ds(LAST_ROW0, ROWS_LAST)])


def _count_pass(dst, zeros_n16):
    k = pl.kernel(
        _count_body,
        out_type=jax.ShapeDtypeStruct((NUM_SC, N, 16), jnp.float32),
        mesh=_sc_mesh,
        scratch_types=(
            [pltpu.VMEM((CHUNK,), jnp.int32)] * NBUF
            + [pltpu.VMEM((CHUNK, 16), jnp.float32)]
            + [pltpu.VMEM_SHARED((N, 16), jnp.float32)]
            + [pltpu.SemaphoreType.DMA] * NBUF
        ),
    )
    return k(dst, zeros_n16)


def _dot(a, b):
    return jnp.dot(a, b, preferred_element_type=jnp.float32)


def _in_proj_body(x_ref, w_ref, b_ref, o_ref):
    o_ref[...] = jnp.maximum(_dot(x_ref[...], w_ref[...]) + b_ref[...], 0.0)


def _in_proj(x, w, b):
    return pl.pallas_call(
        _in_proj_body,
        grid=(GRID_N,),
        in_specs=[
            pl.BlockSpec((ROW_BLK, D), lambda i: (i, 0)),
            pl.BlockSpec((D, D), lambda i: (0, 0)),
            pl.BlockSpec((1, D), lambda i: (0, 0)),
        ],
        out_specs=pl.BlockSpec((ROW_BLK, D), lambda i: (i, 0)),
        out_shape=jax.ShapeDtypeStruct((N, D), jnp.float32),
    )(x, w, b)


def _pre_proj_body(h_ref, wd_ref, ws_ref, pd_ref, ps_ref):
    h = h_ref[...]
    pd_ref[...] = _dot(h, wd_ref[...])
    ps_ref[...] = _dot(h, ws_ref[...])


def _pre_proj(h, wd, ws):
    return pl.pallas_call(
        _pre_proj_body,
        grid=(GRID_N,),
        in_specs=[
            pl.BlockSpec((ROW_BLK, D), lambda i: (i, 0)),
            pl.BlockSpec((D, D), lambda i: (0, 0)),
            pl.BlockSpec((D, D), lambda i: (0, 0)),
        ],
        out_specs=[
            pl.BlockSpec((ROW_BLK, D), lambda i: (i, 0)),
            pl.BlockSpec((ROW_BLK, D), lambda i: (i, 0)),
        ],
        out_shape=[
            jax.ShapeDtypeStruct((N, D), jnp.float32),
            jax.ShapeDtypeStruct((N, D), jnp.float32),
        ],
    )(h, wd, ws)


def _ea_proj_body(ea_ref, w_ref, b_ref, o_ref):
    o_ref[...] = _dot(ea_ref[...], w_ref[...]) + b_ref[...]


def _ea_proj(edge_attr, w_e, b1):
    return pl.pallas_call(
        _ea_proj_body,
        grid=(GRID_E,),
        in_specs=[
            pl.BlockSpec((E_BLK, ED), lambda i: (i, 0)),
            pl.BlockSpec((ED, D), lambda i: (0, 0)),
            pl.BlockSpec((1, D), lambda i: (0, 0)),
        ],
        out_specs=pl.BlockSpec((E_BLK, D), lambda i: (i, 0)),
        out_shape=jax.ShapeDtypeStruct((E, D), jnp.float32),
    )(edge_attr, w_e, b1)


def _update_body(a0_ref, a1_ref, c0_ref, c1_ref, h_ref, w2_ref, b2_ref,
                 u1a_ref, u1b_ref, ub1_ref, u2_ref, ub2_ref, g_ref, b_ref,
                 o_ref):
    h = h_ref[...]
    agg_raw = a0_ref[0] + a1_ref[0]
    cnt = c0_ref[...] + c1_ref[...]
    aggr = _dot(agg_raw, w2_ref[...]) + cnt * b2_ref[...]
    t = jnp.maximum(_dot(h, u1a_ref[...]) + _dot(aggr, u1b_ref[...])
                    + ub1_ref[...], 0.0)
    t2 = _dot(t, u2_ref[...]) + ub2_ref[...]
    mu = jnp.mean(t2, axis=1, keepdims=True)
    var = jnp.mean((t2 - mu) ** 2, axis=1, keepdims=True)
    y = (t2 - mu) * lax.rsqrt(var + 1e-5) * g_ref[...] + b_ref[...]
    o_ref[...] = jnp.maximum(y, 0.0) + h


def _update(acc2, cnt0, cnt1, h, w2, b2, u1a, u1b, ub1, u2, ub2, g, b):
    row = lambda i: (i, 0)
    full = lambda i: (0, 0)
    return pl.pallas_call(
        _update_body,
        grid=(GRID_N,),
        in_specs=[
            pl.BlockSpec((1, ROW_BLK, D), lambda i: (0, i, 0)),
            pl.BlockSpec((1, ROW_BLK, D), lambda i: (1, i, 0)),
            pl.BlockSpec((ROW_BLK, 1), row),
            pl.BlockSpec((ROW_BLK, 1), row),
            pl.BlockSpec((ROW_BLK, D), row),
            pl.BlockSpec((D, D), full),
            pl.BlockSpec((1, D), full),
            pl.BlockSpec((D, D), full),
            pl.BlockSpec((D, D), full),
            pl.BlockSpec((1, D), full),
            pl.BlockSpec((D, D), full),
            pl.BlockSpec((1, D), full),
            pl.BlockSpec((1, D), full),
            pl.BlockSpec((1, D), full),
        ],
        out_specs=pl.BlockSpec((ROW_BLK, D), row),
        out_shape=jax.ShapeDtypeStruct((N, D), jnp.float32),
    )(acc2, acc2, cnt0, cnt1, h, w2, b2, u1a, u1b, ub1, u2, ub2, g, b)


def _out_proj_body(h_ref, w_ref, b_ref, o_ref):
    o_ref[...] = _dot(h_ref[...], w_ref[...]) + b_ref[...]


def _out_proj(h, w, b):
    return pl.pallas_call(
        _out_proj_body,
        grid=(GRID_N,),
        in_specs=[
            pl.BlockSpec((ROW_BLK, D), lambda i: (i, 0)),
            pl.BlockSpec((D, D), lambda i: (0, 0)),
            pl.BlockSpec((1, D), lambda i: (0, 0)),
        ],
        out_specs=pl.BlockSpec((ROW_BLK, D), lambda i: (i, 0)),
        out_shape=jax.ShapeDtypeStruct((N, D), jnp.float32),
    )(h, w, b)


def kernel(x, edge_index, edge_attr, in_w, in_b, msg_w1, msg_b1, msg_w2,
           msg_b2, upd_w1, upd_b1, upd_w2, upd_b2, ln_g, ln_b, out_w, out_b):
    src = edge_index[0]
    dst = edge_index[1]

    zeros_nd = jnp.zeros((N, D), jnp.float32)
    zeros_n16 = jnp.zeros((N, 16), jnp.float32)

    h = _in_proj(x, in_w, in_b.reshape(1, D))

    ea_pre = [_ea_proj(edge_attr, msg_w1[i, 2 * D:, :],
                       msg_b1[i].reshape(1, D)) for i in range(L)]

    cnt2 = _count_pass(dst, zeros_n16)
    cnt0 = cnt2[0, :, 0:1]
    cnt1 = cnt2[1, :, 0:1]

    for i in range(L):
        pre_d, pre_s = _pre_proj(h, msg_w1[i, :D, :], msg_w1[i, D:2 * D, :])
        acc2 = _edge_pass(pre_d, pre_s, ea_pre[i], dst, src, zeros_nd)
        h = _update(acc2, cnt0, cnt1, h,
                    msg_w2[i], msg_b2[i].reshape(1, D),
                    upd_w1[i, :D, :], upd_w1[i, D:, :],
                    upd_b1[i].reshape(1, D),
                    upd_w2[i], upd_b2[i].reshape(1, D),
                    ln_g[i].reshape(1, D), ln_b[i].reshape(1, D))

    return _out_proj(h, out_w, out_b.reshape(1, D))
